# baseline (speedup 1.0000x reference)
import functools

import jax
import jax.numpy as jnp
from jax import lax
from jax.experimental import pallas as pl
from jax.experimental.pallas import tpu as pltpu
from jax.experimental.pallas import tpu_sc as plsc

NC, NS = 2, 16
NW = NC * NS
V = 1_000_000
D = 64
L = 128
BATCH, SEQ = 4096, 200
N_IDX = BATCH * SEQ
IDX_ROWS = N_IDX // L
ROWS_PER_W = IDX_ROWS // NW

G = 16384
GSH = 14
A_STEPS = 31
A_LAST = 61
VPAD = A_STEPS * 2 * G

B_CH_ROWS = 5
B_CH = B_CH_ROWS * L
B_NCH = ROWS_PER_W // B_CH_ROWS

C_BB = 128


def _eye(n):
    return (
        lax.broadcasted_iota(jnp.int32, (n, n), 0)
        == lax.broadcasted_iota(jnp.int32, (n, n), 1)
    ).astype(jnp.float32)


def _mxu_t(x):
    return lax.dot_general(
        x,
        _eye(x.shape[0]),
        (((0,), (0,)), ((), ())),
        preferred_element_type=jnp.float32,
    )


def _tab_transpose_kernel(xl_ref, xr_ref, o_ref):
    o_ref[:, 0:D] = _mxu_t(xl_ref[...])
    o_ref[:, D : 2 * D] = _mxu_t(xr_ref[...])


@jax.jit
def _tab_transpose(wt):
    return pl.pallas_call(
        _tab_transpose_kernel,
        grid=(A_STEPS,),
        in_specs=[
            pl.BlockSpec((D, G), lambda i: (0, 2 * i)),
            pl.BlockSpec((D, G), lambda i: (0, jnp.minimum(2 * i + 1, A_LAST))),
        ],
        out_specs=pl.BlockSpec((G, 2 * D), lambda i: (i, 0)),
        out_shape=jax.ShapeDtypeStruct((VPAD // 2, 2 * D), jnp.float32),
    )(wt, wt)


def _out_transpose_kernel(x_ref, o_ref):
    x3 = x_ref[...].reshape(C_BB, SEQ * D // L, L)
    r = lax.dot_general(
        x3,
        _eye(C_BB),
        (((0,), (0,)), ((), ())),
        preferred_element_type=jnp.float32,
    )
    o_ref[...] = r.reshape(SEQ * D, C_BB)


C_HBLK = BATCH // C_BB // 2


@jax.jit
def _out_transpose_h0(out_half):
    rows_per_blk = C_BB * SEQ * D // L
    return pl.pallas_call(
        _out_transpose_kernel,
        grid=(C_HBLK,),
        in_specs=[pl.BlockSpec((rows_per_blk, L), lambda i: (i, 0))],
        out_specs=pl.BlockSpec((SEQ * D, C_BB), lambda i: (0, i)),
        out_shape=jax.ShapeDtypeStruct((SEQ * D, BATCH), jnp.float32),
    )(out_half)


def _out_transpose_h1_kernel(acc_ref, x_ref, o_ref):
    del acc_ref
    _out_transpose_kernel(x_ref, o_ref)


@jax.jit
def _out_transpose_h1(acc, out_half):
    rows_per_blk = C_BB * SEQ * D // L
    return pl.pallas_call(
        _out_transpose_h1_kernel,
        grid=(C_HBLK,),
        in_specs=[
            pl.BlockSpec(memory_space=pl.ANY),
            pl.BlockSpec((rows_per_blk, L), lambda i: (i, 0)),
        ],
        out_specs=pl.BlockSpec((SEQ * D, C_BB), lambda i: (0, i + C_HBLK)),
        out_shape=jax.ShapeDtypeStruct((SEQ * D, BATCH), jnp.float32),
        input_output_aliases={0: 0},
    )(acc, out_half)


def _make_gather(n_rows):
    rows_per_w = n_rows // NW
    n_chunks = rows_per_w // B_CH_ROWS

    @functools.partial(
        pl.kernel,
        mesh=plsc.VectorSubcoreMesh(core_axis_name="c", subcore_axis_name="s"),
        out_type=jax.ShapeDtypeStruct((n_rows * L, D), jnp.float32),
        scratch_types=[
            pltpu.VMEM((rows_per_w, L), jnp.int32),
            pltpu.VMEM((B_CH, D), jnp.float32),
            pltpu.VMEM((B_CH, D), jnp.float32),
            pltpu.SemaphoreType.DMA,
            pltpu.SemaphoreType.DMA,
        ],
        compiler_params=pltpu.CompilerParams(use_tc_tiling_on_sc=False),
    )
    def gather(idx_hbm, tab_hbm, out_hbm, idx_v, g0, g1, gsem0, gsem1):
        w = lax.axis_index("s") * NC + lax.axis_index("c")
        irow0 = w * rows_per_w
        orow0 = w * (rows_per_w * L)

        pltpu.sync_copy(idx_hbm.at[pl.ds(irow0, rows_per_w)], idx_v)

        def fire(cc, gb, sem):
            return [
                pltpu.async_copy(
                    tab_hbm.at[idx_v.at[cc * B_CH_ROWS + j]],
                    gb.at[pl.ds(j * L, L)],
                    sem,
                )
                for j in range(B_CH_ROWS)
            ]

        def body(i, carry):
            h0 = fire(2 * i, g0, gsem0)
            h1 = fire(2 * i + 1, g1, gsem1)
            for cp in h0:
                cp.wait()
            pltpu.sync_copy(g0, out_hbm.at[pl.ds(orow0 + 2 * i * B_CH, B_CH)])
            for cp in h1:
                cp.wait()
            pltpu.sync_copy(
                g1, out_hbm.at[pl.ds(orow0 + (2 * i + 1) * B_CH, B_CH)]
            )
            return carry

        lax.fori_loop(0, n_chunks // 2, body, 0)

    return gather


_gather_half = _make_gather(IDX_ROWS // 2)


def kernel(input_ids, word_embeddings):
    v = input_ids.reshape(IDX_ROWS, L).astype(jnp.int32)
    g = v >> GSH
    j = v & (G - 1)
    q = ((g >> 1) << (GSH + 1)) + (j << 1) + (g & 1)
    tab_c = _tab_transpose(word_embeddings.T)
    tab2 = tab_c.reshape(VPAD, D)
    h = IDX_ROWS // 2
    out_c1 = _gather_half(q[:h], tab2)
    out_c2 = _gather_half(q[h:], tab2)
    acc = _out_transpose_h0(out_c1.reshape(N_IDX // 4, L))
    out_t = _out_transpose_h1(acc, out_c2.reshape(N_IDX // 4, L))
    return out_t.reshape(SEQ, D, BATCH).transpose(2, 0, 1)


# speedup vs baseline: 1.0028x; 1.0028x over previous
import functools

import jax
import jax.numpy as jnp
from jax import lax
from jax.experimental import pallas as pl
from jax.experimental.pallas import tpu as pltpu
from jax.experimental.pallas import tpu_sc as plsc

NC, NS = 2, 16
NW = NC * NS
V = 1_000_000
D = 64
L = 128
BATCH, SEQ = 4096, 200
N_IDX = BATCH * SEQ
IDX_ROWS = N_IDX // L
ROWS_PER_W = IDX_ROWS // NW

G = 16384
GSH = 14
A_STEPS = 31
A_LAST = 61
VPAD = A_STEPS * 2 * G

B_CH_ROWS = 5
B_CH = B_CH_ROWS * L
B_NCH = ROWS_PER_W // B_CH_ROWS

C_BB = 128


def _eye(n):
    return (
        lax.broadcasted_iota(jnp.int32, (n, n), 0)
        == lax.broadcasted_iota(jnp.int32, (n, n), 1)
    ).astype(jnp.float32)


def _mxu_t(x):
    return lax.dot_general(
        x,
        _eye(x.shape[0]),
        (((0,), (0,)), ((), ())),
        preferred_element_type=jnp.float32,
    )


def _tab_transpose_kernel(xl_ref, xr_ref, o_ref):
    o_ref[:, 0:D] = _mxu_t(xl_ref[...])
    o_ref[:, D : 2 * D] = _mxu_t(xr_ref[...])


@jax.jit
def _tab_transpose(wt):
    return pl.pallas_call(
        _tab_transpose_kernel,
        grid=(A_STEPS,),
        in_specs=[
            pl.BlockSpec((D, G), lambda i: (0, 2 * i)),
            pl.BlockSpec((D, G), lambda i: (0, jnp.minimum(2 * i + 1, A_LAST))),
        ],
        out_specs=pl.BlockSpec((G, 2 * D), lambda i: (i, 0)),
        out_shape=jax.ShapeDtypeStruct((VPAD // 2, 2 * D), jnp.float32),
    )(wt, wt)


def _out_transpose_kernel(x_ref, o_ref):
    x3 = x_ref[...].reshape(C_BB, SEQ * D // L, L)
    r = lax.dot_general(
        x3,
        _eye(C_BB),
        (((0,), (0,)), ((), ())),
        preferred_element_type=jnp.float32,
    )
    o_ref[...] = r.reshape(SEQ * D, C_BB)


@jax.jit
def _out_transpose(out_c2):
    rows_per_blk = C_BB * SEQ * D // L
    return pl.pallas_call(
        _out_transpose_kernel,
        grid=(BATCH // C_BB,),
        in_specs=[pl.BlockSpec((rows_per_blk, L), lambda i: (i, 0))],
        out_specs=pl.BlockSpec((SEQ * D, C_BB), lambda i: (0, i)),
        out_shape=jax.ShapeDtypeStruct((SEQ * D, BATCH), jnp.float32),
    )(out_c2)


def _make_gather(n_rows):
    rows_per_w = n_rows // NW
    n_chunks = rows_per_w // B_CH_ROWS

    @functools.partial(
        pl.kernel,
        mesh=plsc.VectorSubcoreMesh(core_axis_name="c", subcore_axis_name="s"),
        out_type=jax.ShapeDtypeStruct((n_rows * L, D), jnp.float32),
        scratch_types=[
            pltpu.VMEM((rows_per_w, L), jnp.int32),
            pltpu.VMEM((B_CH, D), jnp.float32),
            pltpu.VMEM((B_CH, D), jnp.float32),
            pltpu.SemaphoreType.DMA,
            pltpu.SemaphoreType.DMA,
        ],
        compiler_params=pltpu.CompilerParams(use_tc_tiling_on_sc=False),
    )
    def gather(idx_hbm, tab_hbm, out_hbm, idx_v, g0, g1, gsem0, gsem1):
        w = lax.axis_index("s") * NC + lax.axis_index("c")
        irow0 = w * rows_per_w
        orow0 = w * (rows_per_w * L)

        pltpu.sync_copy(idx_hbm.at[pl.ds(irow0, rows_per_w)], idx_v)

        def fire(cc, gb, sem):
            return [
                pltpu.async_copy(
                    tab_hbm.at[idx_v.at[cc * B_CH_ROWS + j]],
                    gb.at[pl.ds(j * L, L)],
                    sem,
                )
                for j in range(B_CH_ROWS)
            ]

        def body(i, carry):
            h0 = fire(2 * i, g0, gsem0)
            h1 = fire(2 * i + 1, g1, gsem1)
            for cp in h0:
                cp.wait()
            pltpu.sync_copy(g0, out_hbm.at[pl.ds(orow0 + 2 * i * B_CH, B_CH)])
            for cp in h1:
                cp.wait()
            pltpu.sync_copy(
                g1, out_hbm.at[pl.ds(orow0 + (2 * i + 1) * B_CH, B_CH)]
            )
            return carry

        lax.fori_loop(0, n_chunks // 2, body, 0)

    return gather


_gather = _make_gather(IDX_ROWS)


def kernel(input_ids, word_embeddings):
    v = input_ids.reshape(IDX_ROWS, L).astype(jnp.int32)
    g = v >> GSH
    j = v & (G - 1)
    q = ((g >> 1) << (GSH + 1)) + (j << 1) + (g & 1)
    tab_c = _tab_transpose(word_embeddings.T)
    tab2 = tab_c.reshape(VPAD, D)
    out_c = _gather(q, tab2)
    out_t = _out_transpose(out_c.reshape(N_IDX // 2, L))
    return out_t.reshape(SEQ, D, BATCH).transpose(2, 0, 1)


# speedup vs baseline: 1.0042x; 1.0014x over previous
import functools

import jax
import jax.numpy as jnp
from jax import lax
from jax.experimental import pallas as pl
from jax.experimental.pallas import tpu as pltpu
from jax.experimental.pallas import tpu_sc as plsc

NC, NS = 2, 16
NW = NC * NS
V = 1_000_000
D = 64
L = 128
BATCH, SEQ = 4096, 200
N_IDX = BATCH * SEQ
IDX_ROWS = N_IDX // L
ROWS_PER_W = IDX_ROWS // NW

G = 16384
GSH = 14
A_STEPS = 31
A_LAST = 61
VPAD = A_STEPS * 2 * G

B_CH_ROWS = 5
B_CH = B_CH_ROWS * L
B_NCH = ROWS_PER_W // B_CH_ROWS

C_BB = 256


def _eye(n):
    return (
        lax.broadcasted_iota(jnp.int32, (n, n), 0)
        == lax.broadcasted_iota(jnp.int32, (n, n), 1)
    ).astype(jnp.float32)


def _mxu_t(x):
    return lax.dot_general(
        x,
        _eye(x.shape[0]),
        (((0,), (0,)), ((), ())),
        preferred_element_type=jnp.float32,
    )


def _tab_transpose_kernel(xl_ref, xr_ref, o_ref):
    o_ref[:, 0:D] = _mxu_t(xl_ref[...])
    o_ref[:, D : 2 * D] = _mxu_t(xr_ref[...])


@jax.jit
def _tab_transpose(wt):
    return pl.pallas_call(
        _tab_transpose_kernel,
        grid=(A_STEPS,),
        in_specs=[
            pl.BlockSpec((D, G), lambda i: (0, 2 * i)),
            pl.BlockSpec((D, G), lambda i: (0, jnp.minimum(2 * i + 1, A_LAST))),
        ],
        out_specs=pl.BlockSpec((G, 2 * D), lambda i: (i, 0)),
        out_shape=jax.ShapeDtypeStruct((VPAD // 2, 2 * D), jnp.float32),
    )(wt, wt)


def _out_transpose_kernel(x_ref, o_ref):
    x3 = x_ref[...].reshape(C_BB, SEQ * D // L, L)
    r = lax.dot_general(
        x3,
        _eye(C_BB),
        (((0,), (0,)), ((), ())),
        preferred_element_type=jnp.float32,
    )
    o_ref[...] = r.reshape(SEQ * D, C_BB)


@jax.jit
def _out_transpose(out_c2):
    rows_per_blk = C_BB * SEQ * D // L
    return pl.pallas_call(
        _out_transpose_kernel,
        grid=(BATCH // C_BB,),
        in_specs=[pl.BlockSpec((rows_per_blk, L), lambda i: (i, 0))],
        out_specs=pl.BlockSpec((SEQ * D, C_BB), lambda i: (0, i)),
        out_shape=jax.ShapeDtypeStruct((SEQ * D, BATCH), jnp.float32),
    )(out_c2)


def _make_gather(n_rows):
    rows_per_w = n_rows // NW
    n_chunks = rows_per_w // B_CH_ROWS

    @functools.partial(
        pl.kernel,
        mesh=plsc.VectorSubcoreMesh(core_axis_name="c", subcore_axis_name="s"),
        out_type=jax.ShapeDtypeStruct((n_rows * L, D), jnp.float32),
        scratch_types=[
            pltpu.VMEM((rows_per_w, L), jnp.int32),
            pltpu.VMEM((B_CH, D), jnp.float32),
            pltpu.VMEM((B_CH, D), jnp.float32),
            pltpu.SemaphoreType.DMA,
            pltpu.SemaphoreType.DMA,
        ],
        compiler_params=pltpu.CompilerParams(use_tc_tiling_on_sc=False),
    )
    def gather(idx_hbm, tab_hbm, out_hbm, idx_v, g0, g1, gsem0, gsem1):
        w = lax.axis_index("s") * NC + lax.axis_index("c")
        irow0 = w * rows_per_w
        orow0 = w * (rows_per_w * L)

        pltpu.sync_copy(idx_hbm.at[pl.ds(irow0, rows_per_w)], idx_v)

        def fire(cc, gb, sem):
            return [
                pltpu.async_copy(
                    tab_hbm.at[idx_v.at[cc * B_CH_ROWS + j]],
                    gb.at[pl.ds(j * L, L)],
                    sem,
                )
                for j in range(B_CH_ROWS)
            ]

        def body(i, carry):
            h0 = fire(2 * i, g0, gsem0)
            h1 = fire(2 * i + 1, g1, gsem1)
            for cp in h0:
                cp.wait()
            pltpu.sync_copy(g0, out_hbm.at[pl.ds(orow0 + 2 * i * B_CH, B_CH)])
            for cp in h1:
                cp.wait()
            pltpu.sync_copy(
                g1, out_hbm.at[pl.ds(orow0 + (2 * i + 1) * B_CH, B_CH)]
            )
            return carry

        lax.fori_loop(0, n_chunks // 2, body, 0)

    return gather


_gather = _make_gather(IDX_ROWS)


def kernel(input_ids, word_embeddings):
    v = input_ids.reshape(IDX_ROWS, L).astype(jnp.int32)
    g = v >> GSH
    j = v & (G - 1)
    q = ((g >> 1) << (GSH + 1)) + (j << 1) + (g & 1)
    tab_c = _tab_transpose(word_embeddings.T)
    tab2 = tab_c.reshape(VPAD, D)
    out_c = _gather(q, tab2)
    out_t = _out_transpose(out_c.reshape(N_IDX // 2, L))
    return out_t.reshape(SEQ, D, BATCH).transpose(2, 0, 1)
